# baseline (speedup 1.0000x reference)
import functools

import jax
import jax.numpy as jnp
from jax import lax
from jax.experimental import pallas as pl
from jax.experimental.pallas import tpu as pltpu
from jax.experimental.pallas import tpu_sc as plsc

_B = 16384
_K = 20
_D = 64
_NC = 2
_NS = 16
_NW = _NC * _NS
_CPW = _B // _NW
_CH = 64
_NCHUNK = _CPW // _CH
_G = _CH // 16
_NIDX_ROWS = _CH * _K // 128


def _make_sc_kernel():
    mesh = plsc.VectorSubcoreMesh(core_axis_name="c", subcore_axis_name="s")

    @functools.partial(
        pl.kernel,
        mesh=mesh,
        compiler_params=pltpu.CompilerParams(needs_layout_passes=False,
                                             use_tc_tiling_on_sc=False),
        out_type=[
            jax.ShapeDtypeStruct((_B,), jnp.float32),
            jax.ShapeDtypeStruct((_B * _K,), jnp.float32),
        ],
        scratch_types=[
            pltpu.VMEM((_CPW,), jnp.int32),
            pltpu.VMEM((_CPW,), jnp.int32),
            pltpu.VMEM((_CPW * _K // 128, 128), jnp.int32),
            pltpu.VMEM((_CH, _D), jnp.float32),
            pltpu.VMEM((_CH, _D), jnp.float32),
            pltpu.VMEM((_CH * _K, _D), jnp.float32),
            pltpu.VMEM((_CH,), jnp.float32),
            pltpu.VMEM((_CH * _K,), jnp.float32),
            pltpu.SemaphoreType.DMA,
        ],
    )
    def sc_body(tgt_hbm, ctx_hbm, neg_hbm, tw_hbm, cw_hbm,
                pos_hbm, nsc_hbm,
                t_idx_v, c_idx_v, n_idx_v, t_rows, c_rows, n_rows,
                pos_buf, neg_buf, sem):
        wid = lax.axis_index("s") * _NC + lax.axis_index("c")
        lane = lax.iota(jnp.int32, 16)

        wbase = wid * _CPW
        pltpu.sync_copy(tgt_hbm.at[pl.ds(wbase, _CPW)], t_idx_v)
        pltpu.sync_copy(ctx_hbm.at[pl.ds(wbase, _CPW)], c_idx_v)
        nwrows = _CPW * _K // 128
        pltpu.sync_copy(neg_hbm.at[pl.ds(wid * nwrows, nwrows)], n_idx_v)

        def chunk_body(ci, carry):
            base = wbase + ci * _CH

            cps = [
                pltpu.async_copy(tw_hbm.at[t_idx_v.at[pl.ds(ci * _CH, _CH)]],
                                 t_rows, sem),
                pltpu.async_copy(cw_hbm.at[c_idx_v.at[pl.ds(ci * _CH, _CH)]],
                                 c_rows, sem),
            ]
            for j in range(_NIDX_ROWS):
                cps.append(
                    pltpu.async_copy(
                        cw_hbm.at[n_idx_v.at[ci * _NIDX_ROWS + j]],
                        n_rows.at[pl.ds(j * 128, 128)],
                        sem,
                    )
                )
            for cp in cps:
                cp.wait()

            last = lane == 15

            def s_body(s, carry):
                tv = [t_rows[s, pl.ds(i * 16, 16)] for i in range(_D // 16)]
                cv = [c_rows[s, pl.ds(i * 16, 16)] for i in range(_D // 16)]
                acc = tv[0] * cv[0]
                for i in range(1, _D // 16):
                    acc = acc + tv[i] * cv[i]
                plsc.store_scatter(pos_buf, [jnp.full((16,), s, jnp.int32)],
                                   plsc.cumsum(acc), mask=last)
                for k in range(_K):
                    r = s * _K + k
                    nacc = tv[0] * n_rows[r, pl.ds(0, 16)]
                    for i in range(1, _D // 16):
                        nacc = nacc + tv[i] * n_rows[r, pl.ds(i * 16, 16)]
                    plsc.store_scatter(neg_buf,
                                       [jnp.full((16,), r, jnp.int32)],
                                       -plsc.cumsum(nacc), mask=last)
                return carry

            lax.fori_loop(0, _CH, s_body, 0)

            pltpu.sync_copy(pos_buf, pos_hbm.at[pl.ds(base, _CH)])
            pltpu.sync_copy(neg_buf, nsc_hbm.at[pl.ds(base * _K, _CH * _K)])
            return carry

        lax.fori_loop(0, _NCHUNK, chunk_body, 0)

    return sc_body


_SC_KERNEL = _make_sc_kernel()


def kernel(target, context, negatives, target_W, context_W):
    t = target.astype(jnp.int32)
    c = context.astype(jnp.int32)
    n2 = negatives.astype(jnp.int32).reshape(-1, 128)
    pos, neg_flat = _SC_KERNEL(t, c, n2, target_W, context_W)
    return pos, neg_flat.reshape(_B, _K)


# speedup vs baseline: 1.0104x; 1.0104x over previous
import functools

import jax
import jax.numpy as jnp
from jax import lax
from jax.experimental import pallas as pl
from jax.experimental.pallas import tpu as pltpu
from jax.experimental.pallas import tpu_sc as plsc

_B = 16384
_K = 20
_D = 64
_NC = 2
_NS = 16
_NW = _NC * _NS
_CPW = _B // _NW
_CH = 64
_NCHUNK = _CPW // _CH
_PR = 128


def _make_sc_kernel():
    mesh = plsc.VectorSubcoreMesh(core_axis_name="c", subcore_axis_name="s")

    @functools.partial(
        pl.kernel,
        mesh=mesh,
        compiler_params=pltpu.CompilerParams(needs_layout_passes=False,
                                             use_tc_tiling_on_sc=False),
        out_type=[
            jax.ShapeDtypeStruct((_B,), jnp.float32),
            jax.ShapeDtypeStruct((_B * _PR,), jnp.float32),
        ],
        scratch_types=[
            pltpu.VMEM((_CPW,), jnp.int32),
            pltpu.VMEM((_CPW,), jnp.int32),
            pltpu.VMEM((_K, _CPW), jnp.int32),
            pltpu.VMEM((_CH, _D), jnp.float32),
            pltpu.VMEM((_CH, _D), jnp.float32),
            pltpu.VMEM((_CH * _K, _D), jnp.float32),
            pltpu.VMEM((_CH,), jnp.float32),
            pltpu.VMEM((_CH * _PR,), jnp.float32),
            pltpu.SemaphoreType.DMA,
        ],
    )
    def sc_body(tgt_hbm, ctx_hbm, negt_hbm, tw_hbm, cw_hbm,
                pos_hbm, nsc_hbm,
                t_idx_v, c_idx_v, n_idx_v, t_rows, c_rows, n_rows,
                pos_buf, neg_buf, sem):
        wid = lax.axis_index("s") * _NC + lax.axis_index("c")
        lane = lax.iota(jnp.int32, 16)

        wbase = wid * _CPW
        pltpu.sync_copy(tgt_hbm.at[pl.ds(wbase, _CPW)], t_idx_v)
        pltpu.sync_copy(ctx_hbm.at[pl.ds(wbase, _CPW)], c_idx_v)
        pltpu.sync_copy(negt_hbm.at[:, pl.ds(wbase, _CPW)], n_idx_v)

        def chunk_body(ci, carry):
            base = wbase + ci * _CH

            cps = [
                pltpu.async_copy(tw_hbm.at[t_idx_v.at[pl.ds(ci * _CH, _CH)]],
                                 t_rows, sem),
                pltpu.async_copy(cw_hbm.at[c_idx_v.at[pl.ds(ci * _CH, _CH)]],
                                 c_rows, sem),
            ]
            for k in range(_K):
                cps.append(
                    pltpu.async_copy(
                        cw_hbm.at[n_idx_v.at[k, pl.ds(ci * _CH, _CH)]],
                        n_rows.at[pl.ds(k * _CH, _CH)],
                        sem,
                    )
                )
            for cp in cps:
                cp.wait()

            last = lane == 15

            def s_body(s, carry2):
                tv = [t_rows[s, pl.ds(i * 16, 16)] for i in range(_D // 16)]
                cv = [c_rows[s, pl.ds(i * 16, 16)] for i in range(_D // 16)]
                acc = tv[0] * cv[0]
                for i in range(1, _D // 16):
                    acc = acc + tv[i] * cv[i]
                plsc.store_scatter(pos_buf, [jnp.full((16,), s, jnp.int32)],
                                   plsc.cumsum(acc), mask=last)
                for k in range(_K):
                    r = k * _CH + s
                    nacc = tv[0] * n_rows[r, pl.ds(0, 16)]
                    for i in range(1, _D // 16):
                        nacc = nacc + tv[i] * n_rows[r, pl.ds(i * 16, 16)]
                    plsc.store_scatter(neg_buf,
                                       [jnp.full((16,), s * _PR + k, jnp.int32)],
                                       -plsc.cumsum(nacc), mask=last)
                return carry2

            lax.fori_loop(0, _CH, s_body, 0)

            pltpu.sync_copy(pos_buf, pos_hbm.at[pl.ds(base, _CH)])
            pltpu.sync_copy(neg_buf, nsc_hbm.at[pl.ds(base * _PR, _CH * _PR)])
            return carry

        lax.fori_loop(0, _NCHUNK, chunk_body, 0)

    return sc_body


_SC_KERNEL = _make_sc_kernel()


def kernel(target, context, negatives, target_W, context_W):
    t = target.astype(jnp.int32)
    c = context.astype(jnp.int32)
    nt = negatives.astype(jnp.int32).T
    pos, neg_pad = _SC_KERNEL(t, c, nt, target_W, context_W)
    return pos, neg_pad.reshape(_B, _PR)[:, :_K]
